# baseline (speedup 1.0000x reference)
import functools

import jax
import jax.numpy as jnp
from jax import lax
from jax.experimental import pallas as pl
from jax.experimental.pallas import tpu as pltpu
from jax.experimental.pallas import tpu_sc as plsc

N = 10000
E = 320000
KD = 128
MD = 16
NC = 2
NS = 16
NW = NC * NS
EPW = E // NW
NPT = N // NS

def _sc_mesh():
    return plsc.VectorSubcoreMesh(core_axis_name="c", subcore_axis_name="s")


def _rel_dist_sc(pos, src, dst):
    @functools.partial(
        pl.kernel,
        mesh=_sc_mesh(),
        out_type=jax.ShapeDtypeStruct((E,), jnp.float32),
        compiler_params=pltpu.CompilerParams(needs_layout_passes=False),
        scratch_types=[
            pltpu.VMEM((4 * N,), jnp.float32),
            pltpu.VMEM((EPW,), jnp.int32),
            pltpu.VMEM((EPW,), jnp.int32),
            pltpu.VMEM((EPW,), jnp.float32),
        ],
    )
    def k(pos_hbm, src_hbm, dst_hbm, out_hbm, pos_v, src_v, dst_v, rel_v):
        wid = lax.axis_index("s") * NC + lax.axis_index("c")
        base = wid * EPW
        pltpu.sync_copy(pos_hbm, pos_v)
        pltpu.sync_copy(src_hbm.at[pl.ds(base, EPW)], src_v)
        pltpu.sync_copy(dst_hbm.at[pl.ds(base, EPW)], dst_v)

        def body(i, carry):
            s16 = src_v[pl.ds(i * 16, 16)] * 4
            d16 = dst_v[pl.ds(i * 16, 16)] * 4
            dx = plsc.load_gather(pos_v, [s16]) - plsc.load_gather(pos_v, [d16])
            dy = plsc.load_gather(pos_v, [s16 + 1]) - plsc.load_gather(pos_v, [d16 + 1])
            dz = plsc.load_gather(pos_v, [s16 + 2]) - plsc.load_gather(pos_v, [d16 + 2])
            rel_v[pl.ds(i * 16, 16)] = dx * dx + dy * dy + dz * dz
            return carry

        lax.fori_loop(0, EPW // 16, body, 0)
        pltpu.sync_copy(rel_v, out_hbm.at[pl.ds(base, EPW)])

    pos4 = jnp.pad(pos, ((0, 0), (0, 1))).reshape(-1)
    return k(pos4, src, dst)


def _gather_sc(feats, src, dst):
    D = feats.shape[1]
    C = 80

    TT = EPW // C

    @functools.partial(
        pl.kernel,
        mesh=_sc_mesh(),
        out_type=(
            jax.ShapeDtypeStruct((E, D), jnp.bfloat16),
            jax.ShapeDtypeStruct((E, D), jnp.bfloat16),
        ),
        compiler_params=pltpu.CompilerParams(use_tc_tiling_on_sc=False),
        scratch_types=[
            pltpu.VMEM((EPW,), jnp.int32),
            pltpu.VMEM((EPW,), jnp.int32),
            pltpu.VMEM((2, C, D), jnp.bfloat16),
            pltpu.VMEM((2, C, D), jnp.bfloat16),
            pltpu.SemaphoreType.DMA,
            pltpu.SemaphoreType.DMA,
            pltpu.SemaphoreType.DMA,
            pltpu.SemaphoreType.DMA,
        ],
    )
    def k(feats_hbm, src_hbm, dst_hbm, outi_hbm, outj_hbm,
          di_v, si_v, ri_v, rj_v, sem_a_i, sem_a_j, sem_b_i, sem_b_j):
        wid = lax.axis_index("s") * NC + lax.axis_index("c")
        base0 = wid * EPW
        pltpu.sync_copy(dst_hbm.at[pl.ds(base0, EPW)], di_v)
        pltpu.sync_copy(src_hbm.at[pl.ds(base0, EPW)], si_v)

        def start(t, b, sem_i, sem_j):
            cpi = pltpu.async_copy(
                feats_hbm.at[di_v.at[pl.ds(t * C, C)]], ri_v.at[b], sem_i)
            cpj = pltpu.async_copy(
                feats_hbm.at[si_v.at[pl.ds(t * C, C)]], rj_v.at[b], sem_j)
            return cpi, cpj

        def finish(t, b, cps):
            cps[0].wait()
            cps[1].wait()
            pltpu.sync_copy(ri_v.at[b], outi_hbm.at[pl.ds(base0 + t * C, C)])
            pltpu.sync_copy(rj_v.at[b], outj_hbm.at[pl.ds(base0 + t * C, C)])

        start(0, 0, sem_a_i, sem_a_j)

        def body(u, carry):
            t_odd = 2 * u + 1
            cps_b = start(t_odd, 1, sem_b_i, sem_b_j)
            cps_a = (pltpu.make_async_copy(feats_hbm.at[di_v.at[pl.ds(0, C)]], ri_v.at[0], sem_a_i),
                     pltpu.make_async_copy(feats_hbm.at[si_v.at[pl.ds(0, C)]], rj_v.at[0], sem_a_j))
            finish(2 * u, 0, cps_a)
            start(t_odd + 1, 0, sem_a_i, sem_a_j)
            finish(t_odd, 1, cps_b)
            return carry

        lax.fori_loop(0, (TT - 1) // 2, body, 0)
        cps_last = (pltpu.make_async_copy(feats_hbm.at[di_v.at[pl.ds(0, C)]], ri_v.at[0], sem_a_i),
                    pltpu.make_async_copy(feats_hbm.at[si_v.at[pl.ds(0, C)]], rj_v.at[0], sem_a_j))
        finish(TT - 1, 0, cps_last)

    return k(feats, src, dst)


def _scatter_sc(m, dst):
    C = 80
    BIG = 2000
    NBIG = EPW // BIG
    APB = BIG // C

    @functools.partial(
        pl.kernel,
        mesh=_sc_mesh(),
        out_type=jax.ShapeDtypeStruct((2 * N, MD), jnp.float32),
        compiler_params=pltpu.CompilerParams(use_tc_tiling_on_sc=False),
        scratch_types=[
            pltpu.VMEM((EPW // C, C), jnp.int32),
            pltpu.VMEM((2, BIG, MD), jnp.float32),
            pltpu.VMEM((1000, MD), jnp.float32),
            pltpu.VMEM_SHARED((N, MD), jnp.float32),
            pltpu.SemaphoreType.DMA,
            pltpu.SemaphoreType.DMA,
            pltpu.SemaphoreType.DMA,
            pltpu.SemaphoreType.DMA,
        ],
    )
    def k(m_hbm, dst3_hbm, out_hbm, idx_v, mb_v, z_v, acc_sh,
          sem_a, sem_b, sem_add_a, sem_add_b):
        cid = lax.axis_index("c")
        sid = lax.axis_index("s")
        wid = sid * NC + cid
        base0 = wid * EPW

        zero16 = jnp.zeros((16,), jnp.float32)

        @pl.when(sid < 10)
        def _init():
            def zb(i, carry):
                z_v[i, :] = zero16
                return carry

            lax.fori_loop(0, 1000, zb, 0)
            pltpu.sync_copy(z_v, acc_sh.at[pl.ds(sid * 1000, 1000)])

        pltpu.sync_copy(dst3_hbm.at[wid], idx_v)
        plsc.subcore_barrier()

        def load(g, b, sem):
            pltpu.async_copy(m_hbm.at[pl.ds(base0 + g * BIG, BIG)], mb_v.at[b], sem)

        def wait_load(b, sem):
            pltpu.make_async_copy(m_hbm.at[pl.ds(base0, BIG)], mb_v.at[b], sem).wait()

        def adds(g, b, sem_add):
            for j in range(APB):
                pltpu.async_copy(mb_v.at[b, pl.ds(j * C, C)],
                                 acc_sh.at[idx_v.at[g * APB + j]],
                                 sem_add, add=True)

        def drain_adds(b, sem_add):
            for j in range(APB):
                pltpu.make_async_copy(mb_v.at[b, pl.ds(0, C)],
                                      acc_sh.at[idx_v.at[0]], sem_add).wait()

        sems = (sem_a, sem_b)
        add_sems = (sem_add_a, sem_add_b)
        load(0, 0, sem_a)
        for g in range(NBIG):
            b = g % 2
            wait_load(b, sems[b])
            adds(g, b, add_sems[b])
            if g + 1 < NBIG:
                nb = (g + 1) % 2
                if g >= 1:
                    drain_adds(nb, add_sems[nb])
                load(g + 1, nb, sems[nb])
        drain_adds(0, add_sems[0])
        drain_adds(1, add_sems[1])
        plsc.subcore_barrier()

        @pl.when(sid < 10)
        def _out():
            pltpu.sync_copy(acc_sh.at[pl.ds(sid * 1000, 1000)],
                            out_hbm.at[pl.ds(cid * N + sid * 1000, 1000)])

    dst3 = dst.reshape(NW, EPW // C, C)
    return k(m, dst3)


def _silu(x):
    return x * (0.5 * jnp.tanh(x * 0.5) + 0.5)


def _ln(x, g, b):
    mu = jnp.mean(x, axis=-1, keepdims=True)
    xc = x - mu
    var = jnp.mean(xc * xc, axis=-1, keepdims=True)
    return xc * lax.rsqrt(var + 1e-5) * g + b


def _full(shape):
    return pl.BlockSpec(shape, lambda *_: tuple(0 for _ in shape))


def _pre_tc(atom_idx, residue_idx, props, p):
    BN = 1000
    G = N // BN
    pw, pb = p["prop_lin"]["w"], p["prop_lin"]["b"]
    w1, b1 = p["pre1"]["w"], p["pre1"]["b"]
    w2, b2 = p["pre2"]["w"], p["pre2"]["b"]

    def body(a_ref, r_ref, pr_ref, ae_ref, re_ref, pw_ref, pb_ref,
             w1a_ref, w1b_ref, w1c_ref, b1_ref, w2_ref, b2_ref, out_ref, out16_ref):
        a_oh = (lax.broadcasted_iota(jnp.int32, (BN, 22), 1) == a_ref[...]).astype(jnp.float32)
        r_oh = (lax.broadcasted_iota(jnp.int32, (BN, 255), 1) == r_ref[...]).astype(jnp.float32)
        a_emb = jnp.dot(a_oh, ae_ref[...], preferred_element_type=jnp.float32)
        r_emb = jnp.dot(r_oh, re_ref[...], preferred_element_type=jnp.float32)
        pl_ = jnp.dot(pr_ref[...], pw_ref[...], preferred_element_type=jnp.float32) + pb_ref[...]
        h = (jnp.dot(a_emb, w1a_ref[...], preferred_element_type=jnp.float32)
             + jnp.dot(r_emb, w1b_ref[...], preferred_element_type=jnp.float32)
             + jnp.dot(pl_, w1c_ref[...], preferred_element_type=jnp.float32)
             + b1_ref[...])
        h = _silu(h)
        f = jnp.dot(h, w2_ref[...], preferred_element_type=jnp.float32) + b2_ref[...]
        out_ref[...] = f
        out16_ref[...] = f.astype(jnp.bfloat16)

    return pl.pallas_call(
        body,
        grid=(G,),
        in_specs=[
            pl.BlockSpec((BN, 1), lambda i: (i, 0)),
            pl.BlockSpec((BN, 1), lambda i: (i, 0)),
            pl.BlockSpec((BN, 2), lambda i: (i, 0)),
            _full((22, 64)), _full((255, 64)), _full((2, 32)), _full((1, 32)),
            _full((64, KD)), _full((64, KD)), _full((32, KD)), _full((1, KD)),
            _full((KD, KD)), _full((1, KD)),
        ],
        out_specs=(pl.BlockSpec((BN, KD), lambda i: (i, 0)),
                   pl.BlockSpec((BN, KD), lambda i: (i, 0))),
        out_shape=(jax.ShapeDtypeStruct((N, KD), jnp.float32),
                   jax.ShapeDtypeStruct((N, KD), jnp.bfloat16)),
    )(atom_idx, residue_idx, props,
      p["atom_emb"], p["residue_emb"], pw, pb[None, :],
      w1[:64], w1[64:128], w1[128:160], b1[None, :],
      w2, b2[None, :])


def _fourier_tc(rel_t):
    BE = 3200
    G = E // BE

    def body(rel_ref, out_ref):
        inv_scales = jnp.exp2(
            -lax.broadcasted_iota(jnp.int32, (16, 1), 0).astype(jnp.float32))
        rel_b = rel_ref[...]
        xs = inv_scales * rel_b
        out_ref[...] = jnp.concatenate(
            [jnp.sin(xs), jnp.cos(xs), rel_b,
             jnp.ones((1, BE), jnp.float32),
             jnp.zeros((6, BE), jnp.float32)], axis=0)

    return pl.pallas_call(
        body,
        grid=(G,),
        in_specs=[pl.BlockSpec((1, BE), lambda i: (0, i))],
        out_specs=pl.BlockSpec((40, BE), lambda i: (0, i)),
        out_shape=jax.ShapeDtypeStruct((40, E), jnp.float32),
    )(rel_t)


def _edge_tc(xi, xj, four_t, kp):
    BE = 1280
    G = E // BE
    H = 2 * (2 * MD + 1 + 2 * KD)
    w1, b1 = kp["e1"]["w"], kp["e1"]["b"]
    w2, b2 = kp["e2"]["w"], kp["e2"]["b"]
    w1_i = w1[:KD].astype(jnp.bfloat16)
    w1_j = w1[KD:2 * KD].astype(jnp.bfloat16)
    w1_f = jnp.concatenate(
        [w1[2 * KD:], b1[None, :], jnp.zeros((6, w1.shape[1]), jnp.float32)],
        axis=0)

    def body(xi_ref, xj_ref, four_ref, w1i_ref, w1j_ref, w1f_ref,
             w2_ref, b2_ref, g_ref, b_ref, out_ref):
        h = (jnp.dot(xi_ref[...], w1i_ref[...], preferred_element_type=jnp.float32)
             + jnp.dot(xj_ref[...], w1j_ref[...], preferred_element_type=jnp.float32)
             + lax.dot_general(four_ref[...], w1f_ref[...],
                               (((0,), (0,)), ((), ())),
                               preferred_element_type=jnp.float32))
        h = _silu(h)
        m = jnp.dot(h.astype(jnp.bfloat16), w2_ref[...],
                    preferred_element_type=jnp.float32) + b2_ref[...]
        m = _silu(m)
        out_ref[...] = _ln(m, g_ref[...], b_ref[...])

    return pl.pallas_call(
        body,
        grid=(G,),
        in_specs=[
            pl.BlockSpec((BE, KD), lambda i: (i, 0)),
            pl.BlockSpec((BE, KD), lambda i: (i, 0)),
            pl.BlockSpec((40, BE), lambda i: (0, i)),
            _full((KD, H)), _full((KD, H)), _full((40, H)),
            _full((H, MD)), _full((1, MD)),
            _full((1, MD)), _full((1, MD)),
        ],
        out_specs=pl.BlockSpec((BE, MD), lambda i: (i, 0)),
        out_shape=jax.ShapeDtypeStruct((E, MD), jnp.float32),
    )(xi, xj, four_t, w1_i, w1_j, w1_f,
      w2.astype(jnp.bfloat16), b2[None, :],
      kp["ln_e1_g"][None, :], kp["ln_e1_b"][None, :])


def _node_tc(feats, acc, kp):
    BN = 1000
    G = N // BN
    w1, b1 = kp["n1"]["w"], kp["n1"]["b"]
    w2, b2 = kp["n2"]["w"], kp["n2"]["b"]

    def body(f_ref, a0_ref, a1_ref, w1h_ref, w1s_ref, b1_ref, w2_ref, b2_ref,
             ge2_ref, be2_ref, gn1_ref, bn1_ref, gn2_ref, bn2_ref, out_ref, out16_ref):
        feats_b = f_ref[...]
        s = a0_ref[...] + a1_ref[...]
        s = _ln(s, ge2_ref[...], be2_ref[...])
        hidden = _ln(feats_b, gn1_ref[...], bn1_ref[...])
        h = (jnp.dot(hidden, w1h_ref[...], preferred_element_type=jnp.float32)
             + jnp.dot(s, w1s_ref[...], preferred_element_type=jnp.float32)
             + b1_ref[...])
        h = _silu(h)
        ho = jnp.dot(h, w2_ref[...], preferred_element_type=jnp.float32) + b2_ref[...]
        ho = _ln(ho, gn2_ref[...], bn2_ref[...])
        f = feats_b + ho
        out_ref[...] = f
        out16_ref[...] = f.astype(jnp.bfloat16)

    return pl.pallas_call(
        body,
        grid=(G,),
        in_specs=[
            pl.BlockSpec((BN, KD), lambda i: (i, 0)),
            pl.BlockSpec((BN, MD), lambda i: (i, 0)),
            pl.BlockSpec((BN, MD), lambda i: (G + i, 0)),
            _full((KD, 2 * KD)), _full((MD, 2 * KD)), _full((1, 2 * KD)),
            _full((2 * KD, KD)), _full((1, KD)),
            _full((1, MD)), _full((1, MD)), _full((1, KD)), _full((1, KD)),
            _full((1, KD)), _full((1, KD)),
        ],
        out_specs=(pl.BlockSpec((BN, KD), lambda i: (i, 0)),
                   pl.BlockSpec((BN, KD), lambda i: (i, 0))),
        out_shape=(jax.ShapeDtypeStruct((N, KD), jnp.float32),
                   jax.ShapeDtypeStruct((N, KD), jnp.bfloat16)),
    )(feats, acc, acc,
      w1[:KD], w1[KD:], b1[None, :], w2, b2[None, :],
      kp["ln_e2_g"][None, :], kp["ln_e2_b"][None, :],
      kp["ln_n1_g"][None, :], kp["ln_n1_b"][None, :],
      kp["ln_n2_g"][None, :], kp["ln_n2_b"][None, :])


def _post_tc(f1, f2, f3, p):
    BN = 1000
    G = N // BN
    w1, b1 = p["post1"]["w"], p["post1"]["b"]
    w2, b2 = p["post2"]["w"], p["post2"]["b"]
    w3, b3 = p["post3"]["w"], p["post3"]["b"]

    def body(f1_ref, f2_ref, f3_ref, w1a_ref, w1b_ref, w1c_ref, b1_ref,
             w2_ref, b2_ref, w3_ref, b3_ref, out_ref):
        h = (jnp.dot(f1_ref[...], w1a_ref[...], preferred_element_type=jnp.float32)
             + jnp.dot(f2_ref[...], w1b_ref[...], preferred_element_type=jnp.float32)
             + jnp.dot(f3_ref[...], w1c_ref[...], preferred_element_type=jnp.float32)
             + b1_ref[...])
        h = _silu(h)
        h = _silu(jnp.dot(h, w2_ref[...], preferred_element_type=jnp.float32) + b2_ref[...])
        h = _silu(jnp.dot(h, w3_ref[...], preferred_element_type=jnp.float32) + b3_ref[...])
        out_ref[...] = h

    return pl.pallas_call(
        body,
        grid=(G,),
        in_specs=[
            pl.BlockSpec((BN, KD), lambda i: (i, 0)),
            pl.BlockSpec((BN, KD), lambda i: (i, 0)),
            pl.BlockSpec((BN, KD), lambda i: (i, 0)),
            _full((KD, KD)), _full((KD, KD)), _full((KD, KD)), _full((1, KD)),
            _full((KD, KD)), _full((1, KD)), _full((KD, KD)), _full((1, KD)),
        ],
        out_specs=pl.BlockSpec((BN, KD), lambda i: (i, 0)),
        out_shape=jax.ShapeDtypeStruct((N, KD), jnp.float32),
    )(f1, f2, f3, w1[:KD], w1[KD:2 * KD], w1[2 * KD:], b1[None, :],
      w2, b2[None, :], w3, b3[None, :])


def kernel(pos, props, atom_idx, residue_idx, edge_index, params):
    src = edge_index[0]
    dst = edge_index[1]
    rel_t = _rel_dist_sc(pos, src, dst)[None, :]
    four_t = _fourier_tc(rel_t)
    feats, feats16 = _pre_tc(atom_idx[:, None], residue_idx[:, None], props, params)
    feat_list = []
    for kp in params["kernels"]:
        xi, xj = _gather_sc(feats16, src, dst)
        m = _edge_tc(xi, xj, four_t, kp)
        acc = _scatter_sc(m, dst)
        feats, feats16 = _node_tc(feats, acc, kp)
        feat_list.append(feats)
    return _post_tc(feat_list[0], feat_list[1], feat_list[2], params)


# speedup vs baseline: 1.6476x; 1.6476x over previous
import functools

import jax
import jax.numpy as jnp
from jax import lax
from jax.experimental import pallas as pl
from jax.experimental.pallas import tpu as pltpu
from jax.experimental.pallas import tpu_sc as plsc

N = 10000
E = 320000
KD = 128
MD = 16
NC = 2
NS = 16
NW = NC * NS
EPW = E // NW
NPT = N // NS

def _sc_mesh():
    return plsc.VectorSubcoreMesh(core_axis_name="c", subcore_axis_name="s")


def _rel_dist_sc(pos, src, dst):
    @functools.partial(
        pl.kernel,
        mesh=_sc_mesh(),
        out_type=jax.ShapeDtypeStruct((E,), jnp.float32),
        compiler_params=pltpu.CompilerParams(needs_layout_passes=False),
        scratch_types=[
            pltpu.VMEM((4 * N,), jnp.float32),
            pltpu.VMEM((EPW,), jnp.int32),
            pltpu.VMEM((EPW,), jnp.int32),
            pltpu.VMEM((EPW,), jnp.float32),
        ],
    )
    def k(pos_hbm, src_hbm, dst_hbm, out_hbm, pos_v, src_v, dst_v, rel_v):
        wid = lax.axis_index("s") * NC + lax.axis_index("c")
        base = wid * EPW
        pltpu.sync_copy(pos_hbm, pos_v)
        pltpu.sync_copy(src_hbm.at[pl.ds(base, EPW)], src_v)
        pltpu.sync_copy(dst_hbm.at[pl.ds(base, EPW)], dst_v)

        def body(i, carry):
            for u in range(4):
                off = i * 64 + u * 16
                s16 = src_v[pl.ds(off, 16)] * 4
                d16 = dst_v[pl.ds(off, 16)] * 4
                dx = plsc.load_gather(pos_v, [s16]) - plsc.load_gather(pos_v, [d16])
                dy = plsc.load_gather(pos_v, [s16 + 1]) - plsc.load_gather(pos_v, [d16 + 1])
                dz = plsc.load_gather(pos_v, [s16 + 2]) - plsc.load_gather(pos_v, [d16 + 2])
                rel_v[pl.ds(off, 16)] = dx * dx + dy * dy + dz * dz
            return carry

        lax.fori_loop(0, EPW // 64, body, 0)
        pltpu.sync_copy(rel_v, out_hbm.at[pl.ds(base, EPW)])

    pos4 = jnp.pad(pos, ((0, 0), (0, 1))).reshape(-1)
    return k(pos4, src, dst)


def _gather_sc(feats, src, dst):
    D = feats.shape[1]
    C = 80

    TT = EPW // C

    @functools.partial(
        pl.kernel,
        mesh=_sc_mesh(),
        out_type=(
            jax.ShapeDtypeStruct((E, D), jnp.float32),
            jax.ShapeDtypeStruct((E, D), jnp.float32),
        ),
        scratch_types=[
            pltpu.VMEM((EPW,), jnp.int32),
            pltpu.VMEM((EPW,), jnp.int32),
            pltpu.VMEM((2, C, D), jnp.float32),
            pltpu.VMEM((2, C, D), jnp.float32),
            pltpu.SemaphoreType.DMA,
            pltpu.SemaphoreType.DMA,
            pltpu.SemaphoreType.DMA,
            pltpu.SemaphoreType.DMA,
        ],
    )
    def k(feats_hbm, src_hbm, dst_hbm, outi_hbm, outj_hbm,
          di_v, si_v, ri_v, rj_v, sem_a_i, sem_a_j, sem_b_i, sem_b_j):
        wid = lax.axis_index("s") * NC + lax.axis_index("c")
        base0 = wid * EPW
        pltpu.sync_copy(dst_hbm.at[pl.ds(base0, EPW)], di_v)
        pltpu.sync_copy(src_hbm.at[pl.ds(base0, EPW)], si_v)

        def start(t, b, sem_i, sem_j):
            cpi = pltpu.async_copy(
                feats_hbm.at[di_v.at[pl.ds(t * C, C)]], ri_v.at[b], sem_i)
            cpj = pltpu.async_copy(
                feats_hbm.at[si_v.at[pl.ds(t * C, C)]], rj_v.at[b], sem_j)
            return cpi, cpj

        def finish(t, b, cps):
            cps[0].wait()
            cps[1].wait()
            pltpu.sync_copy(ri_v.at[b], outi_hbm.at[pl.ds(base0 + t * C, C)])
            pltpu.sync_copy(rj_v.at[b], outj_hbm.at[pl.ds(base0 + t * C, C)])

        start(0, 0, sem_a_i, sem_a_j)

        def body(u, carry):
            t_odd = 2 * u + 1
            cps_b = start(t_odd, 1, sem_b_i, sem_b_j)
            cps_a = (pltpu.make_async_copy(feats_hbm.at[di_v.at[pl.ds(0, C)]], ri_v.at[0], sem_a_i),
                     pltpu.make_async_copy(feats_hbm.at[si_v.at[pl.ds(0, C)]], rj_v.at[0], sem_a_j))
            finish(2 * u, 0, cps_a)
            start(t_odd + 1, 0, sem_a_i, sem_a_j)
            finish(t_odd, 1, cps_b)
            return carry

        lax.fori_loop(0, (TT - 1) // 2, body, 0)
        cps_last = (pltpu.make_async_copy(feats_hbm.at[di_v.at[pl.ds(0, C)]], ri_v.at[0], sem_a_i),
                    pltpu.make_async_copy(feats_hbm.at[si_v.at[pl.ds(0, C)]], rj_v.at[0], sem_a_j))
        finish(TT - 1, 0, cps_last)

    return k(feats, src, dst)


def _scatter_sc(m, dst):
    C = 80
    BIG = 2000
    NBIG = EPW // BIG
    APB = BIG // C

    @functools.partial(
        pl.kernel,
        mesh=_sc_mesh(),
        out_type=jax.ShapeDtypeStruct((2 * N, MD), jnp.float32),
        compiler_params=pltpu.CompilerParams(use_tc_tiling_on_sc=False),
        scratch_types=[
            pltpu.VMEM((EPW // C, C), jnp.int32),
            pltpu.VMEM((2, BIG, MD), jnp.float32),
            pltpu.VMEM((1000, MD), jnp.float32),
            pltpu.VMEM_SHARED((N, MD), jnp.float32),
            pltpu.SemaphoreType.DMA,
            pltpu.SemaphoreType.DMA,
            pltpu.SemaphoreType.DMA,
            pltpu.SemaphoreType.DMA,
        ],
    )
    def k(m_hbm, dst3_hbm, out_hbm, idx_v, mb_v, z_v, acc_sh,
          sem_a, sem_b, sem_add_a, sem_add_b):
        cid = lax.axis_index("c")
        sid = lax.axis_index("s")
        wid = sid * NC + cid
        base0 = wid * EPW

        zero16 = jnp.zeros((16,), jnp.float32)

        @pl.when(sid < 10)
        def _init():
            def zb(i, carry):
                z_v[i, :] = zero16
                return carry

            lax.fori_loop(0, 1000, zb, 0)
            pltpu.sync_copy(z_v, acc_sh.at[pl.ds(sid * 1000, 1000)])

        pltpu.sync_copy(dst3_hbm.at[wid], idx_v)
        plsc.subcore_barrier()

        def load(g, b, sem):
            pltpu.async_copy(m_hbm.at[pl.ds(base0 + g * BIG, BIG)], mb_v.at[b], sem)

        def wait_load(b, sem):
            pltpu.make_async_copy(m_hbm.at[pl.ds(base0, BIG)], mb_v.at[b], sem).wait()

        def adds(g, b, sem_add):
            for j in range(APB):
                pltpu.async_copy(mb_v.at[b, pl.ds(j * C, C)],
                                 acc_sh.at[idx_v.at[g * APB + j]],
                                 sem_add, add=True)

        def drain_adds(b, sem_add):
            for j in range(APB):
                pltpu.make_async_copy(mb_v.at[b, pl.ds(0, C)],
                                      acc_sh.at[idx_v.at[0]], sem_add).wait()

        sems = (sem_a, sem_b)
        add_sems = (sem_add_a, sem_add_b)
        load(0, 0, sem_a)
        for g in range(NBIG):
            b = g % 2
            wait_load(b, sems[b])
            adds(g, b, add_sems[b])
            if g + 1 < NBIG:
                nb = (g + 1) % 2
                if g >= 1:
                    drain_adds(nb, add_sems[nb])
                load(g + 1, nb, sems[nb])
        drain_adds(0, add_sems[0])
        drain_adds(1, add_sems[1])
        plsc.subcore_barrier()

        @pl.when(sid < 10)
        def _out():
            pltpu.sync_copy(acc_sh.at[pl.ds(sid * 1000, 1000)],
                            out_hbm.at[pl.ds(cid * N + sid * 1000, 1000)])

    dst3 = dst.reshape(NW, EPW // C, C)
    return k(m, dst3)


def _silu(x):
    return x * (0.5 * jnp.tanh(x * 0.5) + 0.5)


def _ln(x, g, b):
    mu = jnp.mean(x, axis=-1, keepdims=True)
    xc = x - mu
    var = jnp.mean(xc * xc, axis=-1, keepdims=True)
    return xc * lax.rsqrt(var + 1e-5) * g + b


def _full(shape):
    return pl.BlockSpec(shape, lambda *_: tuple(0 for _ in shape))


def _pre_tc(atom_idx, residue_idx, props, p):
    BN = 1000
    G = N // BN
    pw, pb = p["prop_lin"]["w"], p["prop_lin"]["b"]
    w1, b1 = p["pre1"]["w"], p["pre1"]["b"]
    w2, b2 = p["pre2"]["w"], p["pre2"]["b"]

    def body(a_ref, r_ref, pr_ref, ae_ref, re_ref, pw_ref, pb_ref,
             w1a_ref, w1b_ref, w1c_ref, b1_ref, w2_ref, b2_ref, out_ref):
        a_oh = (lax.broadcasted_iota(jnp.int32, (BN, 22), 1) == a_ref[...]).astype(jnp.float32)
        r_oh = (lax.broadcasted_iota(jnp.int32, (BN, 255), 1) == r_ref[...]).astype(jnp.float32)
        a_emb = jnp.dot(a_oh, ae_ref[...], preferred_element_type=jnp.float32)
        r_emb = jnp.dot(r_oh, re_ref[...], preferred_element_type=jnp.float32)
        pl_ = jnp.dot(pr_ref[...], pw_ref[...], preferred_element_type=jnp.float32) + pb_ref[...]
        h = (jnp.dot(a_emb, w1a_ref[...], preferred_element_type=jnp.float32)
             + jnp.dot(r_emb, w1b_ref[...], preferred_element_type=jnp.float32)
             + jnp.dot(pl_, w1c_ref[...], preferred_element_type=jnp.float32)
             + b1_ref[...])
        h = _silu(h)
        out_ref[...] = jnp.dot(h, w2_ref[...], preferred_element_type=jnp.float32) + b2_ref[...]

    return pl.pallas_call(
        body,
        grid=(G,),
        in_specs=[
            pl.BlockSpec((BN, 1), lambda i: (i, 0)),
            pl.BlockSpec((BN, 1), lambda i: (i, 0)),
            pl.BlockSpec((BN, 2), lambda i: (i, 0)),
            _full((22, 64)), _full((255, 64)), _full((2, 32)), _full((1, 32)),
            _full((64, KD)), _full((64, KD)), _full((32, KD)), _full((1, KD)),
            _full((KD, KD)), _full((1, KD)),
        ],
        out_specs=pl.BlockSpec((BN, KD), lambda i: (i, 0)),
        out_shape=jax.ShapeDtypeStruct((N, KD), jnp.float32),
    )(atom_idx, residue_idx, props,
      p["atom_emb"], p["residue_emb"], pw, pb[None, :],
      w1[:64], w1[64:128], w1[128:160], b1[None, :],
      w2, b2[None, :])


def _fourier_tc(rel_t):
    BE = 3200
    G = E // BE

    def body(rel_ref, out_ref):
        inv_scales = jnp.exp2(
            -lax.broadcasted_iota(jnp.int32, (16, 1), 0).astype(jnp.float32))
        rel_b = rel_ref[...]
        xs = inv_scales * rel_b
        out_ref[...] = jnp.concatenate(
            [jnp.sin(xs), jnp.cos(xs), rel_b,
             jnp.ones((1, BE), jnp.float32),
             jnp.zeros((6, BE), jnp.float32)], axis=0)

    return pl.pallas_call(
        body,
        grid=(G,),
        in_specs=[pl.BlockSpec((1, BE), lambda i: (0, i))],
        out_specs=pl.BlockSpec((40, BE), lambda i: (0, i)),
        out_shape=jax.ShapeDtypeStruct((40, E), jnp.float32),
    )(rel_t)


def _edge_tc(xi, xj, four_t, kp):
    BE = 1280
    G = E // BE
    H = 2 * (2 * MD + 1 + 2 * KD)
    w1, b1 = kp["e1"]["w"], kp["e1"]["b"]
    w2, b2 = kp["e2"]["w"], kp["e2"]["b"]
    w1_i = w1[:KD].astype(jnp.bfloat16)
    w1_j = w1[KD:2 * KD].astype(jnp.bfloat16)
    w1_f = jnp.concatenate(
        [w1[2 * KD:], b1[None, :], jnp.zeros((6, w1.shape[1]), jnp.float32)],
        axis=0)

    def body(xi_ref, xj_ref, four_ref, w1i_ref, w1j_ref, w1f_ref,
             w2_ref, b2_ref, g_ref, b_ref, out_ref):
        h = (jnp.dot(xi_ref[...].astype(jnp.bfloat16), w1i_ref[...],
                     preferred_element_type=jnp.float32)
             + jnp.dot(xj_ref[...].astype(jnp.bfloat16), w1j_ref[...],
                       preferred_element_type=jnp.float32)
             + lax.dot_general(four_ref[...], w1f_ref[...],
                               (((0,), (0,)), ((), ())),
                               preferred_element_type=jnp.float32))
        h = _silu(h)
        m = jnp.dot(h.astype(jnp.bfloat16), w2_ref[...],
                    preferred_element_type=jnp.float32) + b2_ref[...]
        m = _silu(m)
        out_ref[...] = _ln(m, g_ref[...], b_ref[...])

    return pl.pallas_call(
        body,
        grid=(G,),
        in_specs=[
            pl.BlockSpec((BE, KD), lambda i: (i, 0)),
            pl.BlockSpec((BE, KD), lambda i: (i, 0)),
            pl.BlockSpec((40, BE), lambda i: (0, i)),
            _full((KD, H)), _full((KD, H)), _full((40, H)),
            _full((H, MD)), _full((1, MD)),
            _full((1, MD)), _full((1, MD)),
        ],
        out_specs=pl.BlockSpec((BE, MD), lambda i: (i, 0)),
        out_shape=jax.ShapeDtypeStruct((E, MD), jnp.float32),
    )(xi, xj, four_t, w1_i, w1_j, w1_f,
      w2.astype(jnp.bfloat16), b2[None, :],
      kp["ln_e1_g"][None, :], kp["ln_e1_b"][None, :])


def _node_tc(feats, acc, kp):
    BN = 1000
    G = N // BN
    w1, b1 = kp["n1"]["w"], kp["n1"]["b"]
    w2, b2 = kp["n2"]["w"], kp["n2"]["b"]

    def body(f_ref, a0_ref, a1_ref, w1h_ref, w1s_ref, b1_ref, w2_ref, b2_ref,
             ge2_ref, be2_ref, gn1_ref, bn1_ref, gn2_ref, bn2_ref, out_ref):
        feats_b = f_ref[...]
        s = a0_ref[...] + a1_ref[...]
        s = _ln(s, ge2_ref[...], be2_ref[...])
        hidden = _ln(feats_b, gn1_ref[...], bn1_ref[...])
        h = (jnp.dot(hidden, w1h_ref[...], preferred_element_type=jnp.float32)
             + jnp.dot(s, w1s_ref[...], preferred_element_type=jnp.float32)
             + b1_ref[...])
        h = _silu(h)
        ho = jnp.dot(h, w2_ref[...], preferred_element_type=jnp.float32) + b2_ref[...]
        ho = _ln(ho, gn2_ref[...], bn2_ref[...])
        out_ref[...] = feats_b + ho

    return pl.pallas_call(
        body,
        grid=(G,),
        in_specs=[
            pl.BlockSpec((BN, KD), lambda i: (i, 0)),
            pl.BlockSpec((BN, MD), lambda i: (i, 0)),
            pl.BlockSpec((BN, MD), lambda i: (G + i, 0)),
            _full((KD, 2 * KD)), _full((MD, 2 * KD)), _full((1, 2 * KD)),
            _full((2 * KD, KD)), _full((1, KD)),
            _full((1, MD)), _full((1, MD)), _full((1, KD)), _full((1, KD)),
            _full((1, KD)), _full((1, KD)),
        ],
        out_specs=pl.BlockSpec((BN, KD), lambda i: (i, 0)),
        out_shape=jax.ShapeDtypeStruct((N, KD), jnp.float32),
    )(feats, acc, acc,
      w1[:KD], w1[KD:], b1[None, :], w2, b2[None, :],
      kp["ln_e2_g"][None, :], kp["ln_e2_b"][None, :],
      kp["ln_n1_g"][None, :], kp["ln_n1_b"][None, :],
      kp["ln_n2_g"][None, :], kp["ln_n2_b"][None, :])


def _post_tc(f1, f2, f3, p):
    BN = 1000
    G = N // BN
    w1, b1 = p["post1"]["w"], p["post1"]["b"]
    w2, b2 = p["post2"]["w"], p["post2"]["b"]
    w3, b3 = p["post3"]["w"], p["post3"]["b"]

    def body(f1_ref, f2_ref, f3_ref, w1a_ref, w1b_ref, w1c_ref, b1_ref,
             w2_ref, b2_ref, w3_ref, b3_ref, out_ref):
        h = (jnp.dot(f1_ref[...], w1a_ref[...], preferred_element_type=jnp.float32)
             + jnp.dot(f2_ref[...], w1b_ref[...], preferred_element_type=jnp.float32)
             + jnp.dot(f3_ref[...], w1c_ref[...], preferred_element_type=jnp.float32)
             + b1_ref[...])
        h = _silu(h)
        h = _silu(jnp.dot(h, w2_ref[...], preferred_element_type=jnp.float32) + b2_ref[...])
        h = _silu(jnp.dot(h, w3_ref[...], preferred_element_type=jnp.float32) + b3_ref[...])
        out_ref[...] = h

    return pl.pallas_call(
        body,
        grid=(G,),
        in_specs=[
            pl.BlockSpec((BN, KD), lambda i: (i, 0)),
            pl.BlockSpec((BN, KD), lambda i: (i, 0)),
            pl.BlockSpec((BN, KD), lambda i: (i, 0)),
            _full((KD, KD)), _full((KD, KD)), _full((KD, KD)), _full((1, KD)),
            _full((KD, KD)), _full((1, KD)), _full((KD, KD)), _full((1, KD)),
        ],
        out_specs=pl.BlockSpec((BN, KD), lambda i: (i, 0)),
        out_shape=jax.ShapeDtypeStruct((N, KD), jnp.float32),
    )(f1, f2, f3, w1[:KD], w1[KD:2 * KD], w1[2 * KD:], b1[None, :],
      w2, b2[None, :], w3, b3[None, :])


def kernel(pos, props, atom_idx, residue_idx, edge_index, params):
    src = edge_index[0]
    dst = edge_index[1]
    rel_t = _rel_dist_sc(pos, src, dst)[None, :]
    four_t = _fourier_tc(rel_t)
    feats = _pre_tc(atom_idx[:, None], residue_idx[:, None], props, params)
    feat_list = []
    for kp in params["kernels"]:
        xi, xj = _gather_sc(feats, src, dst)
        m = _edge_tc(xi, xj, four_t, kp)
        acc = _scatter_sc(m, dst)
        feats = _node_tc(feats, acc, kp)
        feat_list.append(feats)
    return _post_tc(feat_list[0], feat_list[1], feat_list[2], params)
